# Initial kernel scaffold; baseline (speedup 1.0000x reference)
import jax
import jax.numpy as jnp
from jax.experimental import pallas as pl


def kernel(edge_index, position, z, batch, solvent, nuc_index, cdft, params):
    raise NotImplementedError("write your pallas kernel here")


# speedup vs baseline: 5.5135x; 5.5135x over previous
import functools

import jax
import jax.numpy as jnp
from jax import lax
from jax.experimental import pallas as pl
from jax.experimental.pallas import tpu as pltpu
from jax.experimental.pallas import tpu_sc as plsc
import numpy as np

N = 10000
E = 320000
G = 512
NF = 128
NBINS = 50
STEP = 0.1
GAMMA = 10.0
LOG2 = float(np.log(2.0))

NC, NS = 2, 16
NW = NC * NS
EPW = E // NW
CHUNK = 80
NCHUNK = EPW // CHUNK
NPAD = 10240
STRIPE = NPAD // NS

NODE_BLK = 1000
EDGE_BLK = 4000


def _ssp(x):
    return jax.nn.softplus(x) - LOG2


def _mesh():
    return plsc.VectorSubcoreMesh(core_axis_name="c", subcore_axis_name="s",
                                  num_cores=NC, num_subcores=NS)


_SC_PARAMS = pltpu.CompilerParams(needs_layout_passes=False)


def _sc_edge_geom(src, dst, px, py, pz, zv):
    @functools.partial(
        pl.kernel,
        out_type=[jax.ShapeDtypeStruct((E,), jnp.float32),
                  jax.ShapeDtypeStruct((E,), jnp.int32)],
        mesh=_mesh(),
        compiler_params=_SC_PARAMS,
        scratch_types=[
            pltpu.VMEM((N,), jnp.float32),
            pltpu.VMEM((N,), jnp.float32),
            pltpu.VMEM((N,), jnp.float32),
            pltpu.VMEM((N,), jnp.int32),
            pltpu.VMEM((EPW,), jnp.int32),
            pltpu.VMEM((EPW,), jnp.int32),
            pltpu.VMEM((EPW,), jnp.float32),
            pltpu.VMEM((EPW,), jnp.int32),
        ],
    )
    def k(src_h, dst_h, px_h, py_h, pz_h, z_h, d2_h, pair_h,
          px_v, py_v, pz_v, z_v, src_v, dst_v, d2_v, pair_v):
        cid = lax.axis_index("c")
        sid = lax.axis_index("s")
        wid = cid * NS + sid
        base = wid * EPW
        pltpu.sync_copy(px_h, px_v)
        pltpu.sync_copy(py_h, py_v)
        pltpu.sync_copy(pz_h, pz_v)
        pltpu.sync_copy(z_h, z_v)
        pltpu.sync_copy(src_h.at[pl.ds(base, EPW)], src_v)
        pltpu.sync_copy(dst_h.at[pl.ds(base, EPW)], dst_v)

        def body(e, _):
            off = e * 16
            s16 = src_v[pl.ds(off, 16)]
            d16 = dst_v[pl.ds(off, 16)]
            dx = (plsc.load_gather(px_v, [d16]) - plsc.load_gather(px_v, [s16]))
            dy = (plsc.load_gather(py_v, [d16]) - plsc.load_gather(py_v, [s16]))
            dz = (plsc.load_gather(pz_v, [d16]) - plsc.load_gather(pz_v, [s16]))
            d2_v[pl.ds(off, 16)] = dx * dx + dy * dy + dz * dz
            zs = plsc.load_gather(z_v, [s16])
            zd = plsc.load_gather(z_v, [d16])
            pair_v[pl.ds(off, 16)] = zs * 100 + zd
            return _

        lax.fori_loop(0, EPW // 16, body, None)
        pltpu.sync_copy(d2_v, d2_h.at[pl.ds(base, EPW)])
        pltpu.sync_copy(pair_v, pair_h.at[pl.ds(base, EPW)])

    return k(src, dst, px, py, pz, zv)


def _sc_gather_rows(pair, src, zz_table, m):
    @functools.partial(
        pl.kernel,
        out_type=[jax.ShapeDtypeStruct((E, NF), jnp.float32),
                  jax.ShapeDtypeStruct((E, NF), jnp.float32)],
        mesh=_mesh(),
        compiler_params=_SC_PARAMS,
        scratch_types=[
            pltpu.VMEM((CHUNK,), jnp.int32),
            pltpu.VMEM((CHUNK,), jnp.int32),
            pltpu.VMEM((CHUNK, NF), jnp.float32),
            pltpu.VMEM((CHUNK, NF), jnp.float32),
            pltpu.SemaphoreType.DMA,
            pltpu.SemaphoreType.DMA,
        ],
    )
    def k(pair_h, src_h, table_h, m_h, zz_out, m_out,
          idx_p, idx_s, rows_a, rows_b, sem_a, sem_b):
        cid = lax.axis_index("c")
        sid = lax.axis_index("s")
        wid = cid * NS + sid
        base = wid * EPW

        def body(c, _):
            off = base + c * CHUNK
            pltpu.sync_copy(pair_h.at[pl.ds(off, CHUNK)], idx_p)
            pltpu.sync_copy(src_h.at[pl.ds(off, CHUNK)], idx_s)
            cp_a = pltpu.async_copy(table_h.at[idx_p], rows_a, sem_a)
            cp_b = pltpu.async_copy(m_h.at[idx_s], rows_b, sem_b)
            cp_a.wait()
            cp_b.wait()
            pltpu.sync_copy(rows_a, zz_out.at[pl.ds(off, CHUNK)])
            pltpu.sync_copy(rows_b, m_out.at[pl.ds(off, CHUNK)])
            return _

        lax.fori_loop(0, NCHUNK, body, None)

    return k(pair, src, zz_table, m)


def _sc_scatter_add(out_e, dst, zeros):
    @functools.partial(
        pl.kernel,
        out_type=[jax.ShapeDtypeStruct((NPAD, NF), jnp.float32),
                  jax.ShapeDtypeStruct((NPAD, NF), jnp.float32)],
        mesh=_mesh(),
        compiler_params=_SC_PARAMS,
        scratch_types=[
            pltpu.VMEM_SHARED((NPAD, NF), jnp.float32),
            pltpu.VMEM((CHUNK,), jnp.int32),
            pltpu.VMEM((CHUNK, NF), jnp.float32),
        ],
    )
    def k(oute_h, dst_h, zeros_h, vh0, vh1, accum, idx_d, rows):
        cid = lax.axis_index("c")
        sid = lax.axis_index("s")
        wid = cid * NS + sid
        base = wid * EPW
        pltpu.sync_copy(zeros_h.at[pl.ds(sid * STRIPE, STRIPE)],
                        accum.at[pl.ds(sid * STRIPE, STRIPE)])
        plsc.subcore_barrier()

        def body(c, _):
            off = base + c * CHUNK
            pltpu.sync_copy(dst_h.at[pl.ds(off, CHUNK)], idx_d)
            pltpu.sync_copy(oute_h.at[pl.ds(off, CHUNK)], rows)
            pltpu.sync_copy(rows, accum.at[idx_d], add=True)
            return _

        lax.fori_loop(0, NCHUNK, body, None)
        plsc.subcore_barrier()

        @pl.when(cid == 0)
        def _():
            pltpu.sync_copy(accum.at[pl.ds(sid * STRIPE, STRIPE)],
                            vh0.at[pl.ds(sid * STRIPE, STRIPE)])

        @pl.when(cid == 1)
        def _():
            pltpu.sync_copy(accum.at[pl.ds(sid * STRIPE, STRIPE)],
                            vh1.at[pl.ds(sid * STRIPE, STRIPE)])

    return k(out_e, dst, zeros)


def _tc_embed(z2d, batch2d, emb_z):
    nb = N // NODE_BLK

    def body(z_ref, b_ref, emb_ref, x_ref, cnt_ref):
        i = pl.program_id(0)
        oh = (z_ref[...] == lax.broadcasted_iota(jnp.int32, (NODE_BLK, 100), 1)
              ).astype(jnp.float32)
        x_ref[...] = jnp.dot(oh, emb_ref[...], preferred_element_type=jnp.float32)
        ohb = (b_ref[...] == lax.broadcasted_iota(jnp.int32, (NODE_BLK, G), 1)
               ).astype(jnp.float32)
        cnt = lax.dot_general(ohb, jnp.ones((NODE_BLK, 1), jnp.float32),
                              (((0,), (0,)), ((), ())),
                              preferred_element_type=jnp.float32)

        @pl.when(i == 0)
        def _():
            cnt_ref[...] = jnp.zeros_like(cnt_ref)

        cnt_ref[...] += cnt

    return pl.pallas_call(
        body,
        grid=(nb,),
        in_specs=[pl.BlockSpec((NODE_BLK, 1), lambda i: (i, 0)),
                  pl.BlockSpec((NODE_BLK, 1), lambda i: (i, 0)),
                  pl.BlockSpec((100, NF), lambda i: (0, 0))],
        out_specs=[pl.BlockSpec((NODE_BLK, NF), lambda i: (i, 0)),
                   pl.BlockSpec((G, 1), lambda i: (0, 0))],
        out_shape=[jax.ShapeDtypeStruct((N, NF), jnp.float32),
                   jax.ShapeDtypeStruct((G, 1), jnp.float32)],
    )(z2d, batch2d, emb_z)


def _tc_zz_tables(cf0, g0, cf1, g1):
    npair = 100 * 100
    blk = 1000
    w00, b00, w01, b01 = g0
    w10, b10, w11, b11 = g1

    def body(cf0_r, w00_r, b00_r, w01_r, b01_r,
             cf1_r, w10_r, b10_r, w11_r, b11_r, t0_r, t1_r):
        i = pl.program_id(0)
        r = lax.broadcasted_iota(jnp.int32, (blk, 1), 0)
        a = i * (blk // 100) + r // 100
        b = r % 100
        col = lax.broadcasted_iota(jnp.int32, (blk, 100), 1)
        oha = (a == col).astype(jnp.float32)
        ohb = (b == col).astype(jnp.float32)
        for cf_r, w0_r, bb0_r, w1_r, bb1_r, t_r in (
                (cf0_r, w00_r, b00_r, w01_r, b01_r, t0_r),
                (cf1_r, w10_r, b10_r, w11_r, b11_r, t1_r)):
            za = jnp.dot(oha, cf_r[...], preferred_element_type=jnp.float32)
            zb = jnp.dot(ohb, cf_r[...], preferred_element_type=jnp.float32)
            h = _ssp(jnp.dot(za * zb, w0_r[...],
                             preferred_element_type=jnp.float32) + bb0_r[...])
            t_r[...] = _ssp(jnp.dot(h, w1_r[...],
                                    preferred_element_type=jnp.float32) + bb1_r[...])

    full = lambda s: pl.BlockSpec(s, lambda i: tuple(0 for _ in s))
    in_specs = [full((100, NF)), full((NF, NF)), full((1, NF)), full((NF, NF)),
                full((1, NF)),
                full((100, NF)), full((NF, NF)), full((1, NF)), full((NF, NF)),
                full((1, NF))]
    return pl.pallas_call(
        body,
        grid=(npair // blk,),
        in_specs=in_specs,
        out_specs=[pl.BlockSpec((blk, NF), lambda i: (i, 0)),
                   pl.BlockSpec((blk, NF), lambda i: (i, 0))],
        out_shape=[jax.ShapeDtypeStruct((npair, NF), jnp.float32),
                   jax.ShapeDtypeStruct((npair, NF), jnp.float32)],
    )(cf0, w00, b00, w01, b01, cf1, w10, b10, w11, b11)


def _tc_small_heads(sol2d, emb_solv, ps, cdft, pu):
    ws0, bs0, ws1, bs1 = ps
    wu0, bu0, wu1, bu1 = pu

    def body(sol_r, es_r, ws0_r, bs0_r, ws1_r, bs1_r,
             cd_r, wu0_r, bu0_r, wu1_r, bu1_r, sv_r, u0_r):
        oh = (sol_r[...] == lax.broadcasted_iota(jnp.int32, (G, 4), 1)
              ).astype(jnp.float32)
        se = jnp.dot(oh, es_r[...], preferred_element_type=jnp.float32)
        h = _ssp(jnp.dot(se, ws0_r[...], preferred_element_type=jnp.float32)
                 + bs0_r[...])
        sv_r[...] = jnp.dot(h, ws1_r[...], preferred_element_type=jnp.float32) \
            + bs1_r[...]
        hu = _ssp(jnp.dot(cd_r[...], wu0_r[...],
                          preferred_element_type=jnp.float32) + bu0_r[...])
        u0_r[...] = jnp.dot(hu, wu1_r[...], preferred_element_type=jnp.float32) \
            + bu1_r[...]

    return pl.pallas_call(
        body,
        out_shape=[jax.ShapeDtypeStruct((G, 32), jnp.float32),
                   jax.ShapeDtypeStruct((G, NF), jnp.float32)],
    )(sol2d, emb_solv, ws0, bs0, ws1, bs1, cdft, wu0, bu0, wu1, bu1)


def _tc_lin1(x, w, b):
    nb = N // NODE_BLK

    def body(x_r, w_r, b_r, o_r):
        o_r[...] = jnp.dot(x_r[...], w_r[...],
                           preferred_element_type=jnp.float32) + b_r[...]

    return pl.pallas_call(
        body,
        grid=(nb,),
        in_specs=[pl.BlockSpec((NODE_BLK, NF), lambda i: (i, 0)),
                  pl.BlockSpec((NF, NF), lambda i: (0, 0)),
                  pl.BlockSpec((1, NF), lambda i: (0, 0))],
        out_specs=pl.BlockSpec((NODE_BLK, NF), lambda i: (i, 0)),
        out_shape=jax.ShapeDtypeStruct((N, NF), jnp.float32),
    )(x, w, b)


def _tc_edge_mlp(d2e, zz_e, m_e, pg):
    wg0, bg0, wg1, bg1 = pg
    nb = E // EDGE_BLK

    def body(d2_r, zz_r, m_r, wg0_r, bg0_r, wg1_r, bg1_r, o_r):
        d = jnp.sqrt(d2_r[...])
        uk = lax.broadcasted_iota(jnp.int32, (EDGE_BLK, NBINS), 1
                                  ).astype(jnp.float32) * STEP
        g = jnp.exp(-GAMMA * jnp.square(d - uk))
        g = _ssp(jnp.dot(g, wg0_r[...], preferred_element_type=jnp.float32)
                 + bg0_r[...])
        g = _ssp(jnp.dot(g, wg1_r[...], preferred_element_type=jnp.float32)
                 + bg1_r[...])
        o_r[...] = g * zz_r[...] * m_r[...]

    return pl.pallas_call(
        body,
        grid=(nb,),
        in_specs=[pl.BlockSpec((EDGE_BLK, 1), lambda i: (i, 0)),
                  pl.BlockSpec((EDGE_BLK, NF), lambda i: (i, 0)),
                  pl.BlockSpec((EDGE_BLK, NF), lambda i: (i, 0)),
                  pl.BlockSpec((NBINS, NF), lambda i: (0, 0)),
                  pl.BlockSpec((1, NF), lambda i: (0, 0)),
                  pl.BlockSpec((NF, NF), lambda i: (0, 0)),
                  pl.BlockSpec((1, NF), lambda i: (0, 0))],
        out_specs=pl.BlockSpec((EDGE_BLK, NF), lambda i: (i, 0)),
        out_shape=jax.ShapeDtypeStruct((E, NF), jnp.float32),
    )(d2e, zz_e, m_e, wg0, bg0, wg1, bg1)


def _tc_update(vh0, vh1, x, batch2d, pm):
    wm0, bm0, wm1, bm1 = pm
    nb = N // NODE_BLK

    def body(v0_r, v1_r, x_r, b_r, wm0_r, bm0_r, wm1_r, bm1_r, xn_r, s_r):
        i = pl.program_id(0)
        v = v0_r[...] + v1_r[...]
        h = _ssp(jnp.dot(v, wm0_r[...], preferred_element_type=jnp.float32)
                 + bm0_r[...])
        v2 = jnp.dot(h, wm1_r[...], preferred_element_type=jnp.float32) \
            + bm1_r[...]
        xn = x_r[...] + v2
        xn_r[...] = xn
        ohb = (b_r[...] == lax.broadcasted_iota(jnp.int32, (NODE_BLK, G), 1)
               ).astype(jnp.float32)
        contrib = lax.dot_general(ohb, xn, (((0,), (0,)), ((), ())),
                                  preferred_element_type=jnp.float32)

        @pl.when(i == 0)
        def _():
            s_r[...] = jnp.zeros_like(s_r)

        s_r[...] += contrib

    return pl.pallas_call(
        body,
        grid=(nb,),
        in_specs=[pl.BlockSpec((NODE_BLK, NF), lambda i: (i, 0)),
                  pl.BlockSpec((NODE_BLK, NF), lambda i: (i, 0)),
                  pl.BlockSpec((NODE_BLK, NF), lambda i: (i, 0)),
                  pl.BlockSpec((NODE_BLK, 1), lambda i: (i, 0)),
                  pl.BlockSpec((NF, NF), lambda i: (0, 0)),
                  pl.BlockSpec((1, NF), lambda i: (0, 0)),
                  pl.BlockSpec((NF, NF), lambda i: (0, 0)),
                  pl.BlockSpec((1, NF), lambda i: (0, 0))],
        out_specs=[pl.BlockSpec((NODE_BLK, NF), lambda i: (i, 0)),
                   pl.BlockSpec((G, NF), lambda i: (0, 0))],
        out_shape=[jax.ShapeDtypeStruct((N, NF), jnp.float32),
                   jax.ShapeDtypeStruct((G, NF), jnp.float32)],
    )(vh0, vh1, x, batch2d, wm0, bm0, wm1, bm1)


def _tc_u_update(sums, cnt, u, p1, p2, p3):
    w1a, b1a, w1b, b1b = p1
    w2a, b2a, w2b, b2b = p2
    w3a, b3a, w3b, b3b = p3

    def mid(x, wa, ba, wb, bb):
        h = _ssp(jnp.dot(x, wa, preferred_element_type=jnp.float32) + ba)
        return jnp.dot(h, wb, preferred_element_type=jnp.float32) + bb

    def body(s_r, c_r, u_r, w1a_r, b1a_r, w1b_r, b1b_r,
             w2a_r, b2a_r, w2b_r, b2b_r, w3a_r, b3a_r, w3b_r, b3b_r, o_r):
        mean = s_r[...] / jnp.maximum(c_r[...], 1.0)
        u = u_r[...]
        m = mid(mean, w1a_r[...], b1a_r[...], w1b_r[...], b1b_r[...]) \
            + mid(u, w2a_r[...], b2a_r[...], w2b_r[...], b2b_r[...])
        m = mid(m, w3a_r[...], b3a_r[...], w3b_r[...], b3b_r[...])
        o_r[...] = u + m

    return pl.pallas_call(
        body,
        out_shape=jax.ShapeDtypeStruct((G, NF), jnp.float32),
    )(sums, cnt, u, w1a, b1a, w1b, b1b, w2a, b2a, w2b, b2b, w3a, b3a, w3b, b3b)


def _tc_pick_rows(idx2d, x):
    nb = N // NODE_BLK

    def body(i_r, x_r, o_r):
        i = pl.program_id(0)
        col = lax.broadcasted_iota(jnp.int32, (G, NODE_BLK), 1) + i * NODE_BLK
        oh = (i_r[...] == col).astype(jnp.float32)
        contrib = lax.dot_general(oh, x_r[...], (((1,), (0,)), ((), ())),
                                  preferred_element_type=jnp.float32)

        @pl.when(i == 0)
        def _():
            o_r[...] = jnp.zeros_like(o_r)

        o_r[...] += contrib

    return pl.pallas_call(
        body,
        grid=(nb,),
        in_specs=[pl.BlockSpec((G, 1), lambda i: (0, 0)),
                  pl.BlockSpec((NODE_BLK, NF), lambda i: (i, 0))],
        out_specs=pl.BlockSpec((G, NF), lambda i: (0, 0)),
        out_shape=jax.ShapeDtypeStruct((G, NF), jnp.float32),
    )(idx2d, x)


def _tc_final(xs, u, sv, w0a, w0b, w0c, b0, w1, b1, w2, b2):
    def body(xs_r, u_r, sv_r, w0a_r, w0b_r, w0c_r, b0_r, w1_r, b1_r,
             w2_r, b2_r, o_r):
        h = _ssp(jnp.dot(xs_r[...], w0a_r[...], preferred_element_type=jnp.float32)
                 + jnp.dot(u_r[...], w0b_r[...], preferred_element_type=jnp.float32)
                 + jnp.dot(sv_r[...], w0c_r[...], preferred_element_type=jnp.float32)
                 + b0_r[...])
        h = _ssp(jnp.dot(h, w1_r[...], preferred_element_type=jnp.float32)
                 + b1_r[...])
        o_r[...] = jnp.dot(h, w2_r[...], preferred_element_type=jnp.float32) \
            + b2_r[...]

    return pl.pallas_call(
        body,
        out_shape=jax.ShapeDtypeStruct((G, 1), jnp.float32),
    )(xs, u, sv, w0a, w0b, w0c, b0, w1, b1, w2, b2)


def _row(b):
    return b.reshape(1, -1)


def kernel(edge_index, position, z, batch, solvent, nuc_index, cdft, params):
    src = edge_index[0]
    dst = edge_index[1]
    px = jnp.asarray(position[:, 0])
    py = jnp.asarray(position[:, 1])
    pz = jnp.asarray(position[:, 2])
    z2d = z.reshape(N, 1)
    batch2d = batch.reshape(N, 1)
    sol2d = solvent.reshape(G, 1)
    zeros = jnp.zeros((NPAD, NF), jnp.float32)

    p = params
    convs = p['convs']

    def lin4(ps):
        (wa, ba), (wb, bb) = ps
        return (wa, _row(ba), wb, _row(bb))

    d2, pair = _sc_edge_geom(src, dst, px, py, pz, z)
    d2e = d2.reshape(E, 1)

    x, cnt = _tc_embed(z2d, batch2d, p['emb_z'])
    t0, t1 = _tc_zz_tables(convs[0]['cf_emb_z'], lin4(convs[0]['mlp_z']),
                           convs[1]['cf_emb_z'], lin4(convs[1]['mlp_z']))
    sv, u = _tc_small_heads(sol2d, p['emb_solv'], lin4(p['mlp_solv']),
                            cdft, lin4(p['mlp_u0']))

    tables = (t0, t1)
    for ci, cp in enumerate(convs):
        w1, b1 = cp['lin1']
        m = _tc_lin1(x, w1, _row(b1))
        zz_e, m_e = _sc_gather_rows(pair, src, tables[ci], m)
        out_e = _tc_edge_mlp(d2e, zz_e, m_e, lin4(cp['mlp_g']))
        vh0, vh1 = _sc_scatter_add(out_e, dst, zeros)
        x, sums = _tc_update(vh0, vh1, x, batch2d, lin4(cp['mlp']))
        u = _tc_u_update(sums, cnt, u, lin4(p['mlp_u1']), lin4(p['mlp_u2']),
                         lin4(p['mlp_u3']))

    idx2d = ((nuc_index - 1) % N).reshape(G, 1).astype(jnp.int32)
    xs = _tc_pick_rows(idx2d, x)

    (w0, b0), (wp1, bp1), (wp2, bp2) = p['post']
    out = _tc_final(xs, u, sv, w0[:NF], w0[NF:2 * NF], w0[2 * NF:], _row(b0),
                    wp1, _row(bp1), wp2, _row(bp2))
    return out
